# baseline (speedup 1.0000x reference)
import functools

import numpy as np
import jax
import jax.numpy as jnp
from jax import lax
from jax.experimental import pallas as pl
from jax.experimental.pallas import tpu as pltpu
from jax.experimental.pallas import tpu_sc as plsc

B, H, N, F = 8, 12, 576, 768
G = 16
L = 16
NC, NS = 2, 16
NW = NC * NS
JCH = N // L
NEG = np.float32(-3.0e38)

HSC = 5
HTC = H - HSC
RCH = 72
RPS = N // RCH
NSLAB = B * HSC
NIT = NSLAB * RPS
IPT = NIT // NW
JB = 12

CH = 1
NBUF = 8


def _sc_reduce(attn):
    mesh = plsc.VectorSubcoreMesh(core_axis_name="c", subcore_axis_name="s")

    @functools.partial(
        pl.kernel,
        out_type=jax.ShapeDtypeStruct((B, NW, N), jnp.float32),
        mesh=mesh,
        scratch_types=[
            pltpu.VMEM((2, RCH, N), jnp.float32),
            pltpu.VMEM((B, N), jnp.float32),
            pltpu.SemaphoreType.DMA((2,)),
        ],
    )
    def k(a_hbm, parts_hbm, buf, acc, sems):
        wid = lax.axis_index("s") * NC + lax.axis_index("c")
        zero = jnp.zeros((L,), jnp.float32)
        for bb in range(B):
            for j in range(JCH):
                acc[bb, pl.ds(j * L, L)] = zero

        def copy_item(q, parity):
            it = wid + q * NW
            s = it // RPS
            c = it - s * RPS
            bs = s // HSC
            hs = HTC + (s - bs * HSC)
            return pltpu.make_async_copy(
                a_hbm.at[bs, hs, pl.ds(c * RCH, RCH)],
                buf.at[parity], sems.at[parity])

        copy_item(0, 0).start()
        for q in range(IPT):
            parity = q % 2
            if q + 1 < IPT:
                copy_item(q + 1, 1 - parity).start()
            copy_item(q, parity).wait()
            it = wid + q * NW
            b = it // (HSC * RPS)
            for j0 in range(0, JCH, JB):
                def body(r, accs):
                    return tuple(
                        a + buf[parity, r, pl.ds((j0 + j) * L, L)]
                        for j, a in enumerate(accs))

                accs = lax.fori_loop(0, RCH, body,
                                     tuple(zero for _ in range(JB)))
                for j, a in enumerate(accs):
                    off = (j0 + j) * L
                    acc[b, pl.ds(off, L)] = acc[b, pl.ds(off, L)] + a
        pltpu.sync_copy(acc, parts_hbm.at[:, wid])

    return k(attn)


def _tc_scores(attn):
    hpc = HTC // CH

    def body(a_hbm, o_ref, bufs, sems):
        def start(b, hh):
            slot = (b * hpc + hh) % NBUF
            pltpu.make_async_copy(a_hbm.at[b, pl.ds(hh * CH, CH)],
                                  bufs.at[slot], sems.at[slot]).start()

        for i in range(NBUF):
            start(i // hpc, i % hpc)
        for b in range(B):
            acc = jnp.zeros((N,), jnp.float32)
            for hh in range(hpc):
                i = b * hpc + hh
                slot = i % NBUF
                pltpu.make_async_copy(a_hbm.at[b, pl.ds(hh * CH, CH)],
                                      bufs.at[slot], sems.at[slot]).wait()
                acc = acc + jnp.sum(bufs[slot], axis=(0, 1))
                if i + NBUF < B * hpc:
                    j = i + NBUF
                    start(j // hpc, j % hpc)
            o_ref[b] = acc

    return pl.pallas_call(
        body,
        in_specs=[pl.BlockSpec(memory_space=pltpu.HBM)],
        out_specs=pl.BlockSpec(memory_space=pltpu.VMEM),
        out_shape=jax.ShapeDtypeStruct((B, N), jnp.float32),
        scratch_shapes=[pltpu.VMEM((NBUF, CH, N, N), jnp.float32),
                        pltpu.SemaphoreType.DMA((NBUF,))],
    )(attn)


def _topk_gather(tc_scores, parts, x2):
    mesh = plsc.VectorSubcoreMesh(core_axis_name="c", subcore_axis_name="s")

    @functools.partial(
        pl.kernel,
        out_type=jax.ShapeDtypeStruct((B * G, F), jnp.float32),
        mesh=mesh,
        scratch_types=[
            pltpu.VMEM((N,), jnp.float32),
            pltpu.VMEM((NW, N), jnp.float32),
            pltpu.VMEM((G,), jnp.int32),
            pltpu.VMEM((G, F), jnp.float32),
            pltpu.SemaphoreType.DMA,
        ],
    )
    def k(tc_hbm, parts_hbm, x_hbm, out_hbm, s_v, p_v, idx_v, rows_v, sem):
        wid = lax.axis_index("s") * NC + lax.axis_index("c")

        @pl.when(wid < B)
        def _():
            b = wid
            pltpu.sync_copy(tc_hbm.at[b], s_v)
            pltpu.sync_copy(parts_hbm.at[b], p_v)
            lanes = lax.iota(jnp.int32, L)

            for j0 in range(0, JCH, JB):
                def body(t, accs):
                    return tuple(a + p_v[t, pl.ds((j0 + j) * L, L)]
                                 for j, a in enumerate(accs))

                accs = lax.fori_loop(
                    0, NW, body,
                    tuple(s_v[pl.ds((j0 + j) * L, L)] for j in range(JB)))
                for j, a in enumerate(accs):
                    s_v[pl.ds((j0 + j) * L, L)] = a

            def outer(k_i, topk):
                def scan(j, c):
                    bv, bi = c
                    v = s_v[pl.ds(j * L, L)]
                    take = v > bv
                    return (jnp.where(take, v, bv),
                            jnp.where(take, j * L + lanes, bi))

                bv, bi = lax.fori_loop(
                    0, JCH, scan,
                    (jnp.full((L,), NEG, jnp.float32),
                     jnp.zeros((L,), jnp.int32)))
                best, besti = bv[0], bi[0]
                for i in range(1, L):
                    vi, ni = bv[i], bi[i]
                    upd = (vi > best) | ((vi == best) & (ni < besti))
                    best = jnp.where(upd, vi, best)
                    besti = jnp.where(upd, ni, besti)
                cb = (besti // L) * L
                cur = s_v[pl.ds(cb, L)]
                s_v[pl.ds(cb, L)] = jnp.where(lanes == besti - cb, NEG, cur)
                return jnp.where(lanes == k_i, besti, topk)

            topk = lax.fori_loop(0, G, outer, jnp.zeros((L,), jnp.int32))
            idx_v[...] = topk + b * N
            pltpu.async_copy(x_hbm.at[idx_v], rows_v, sem).wait()
            pltpu.sync_copy(rows_v, out_hbm.at[pl.ds(b * G, G)])

    return k(tc_scores, parts, x2)


def kernel(x, attn_weights):
    parts = _sc_reduce(attn_weights)
    tc = _tc_scores(attn_weights)
    out = _topk_gather(tc, parts, x.reshape(B * N, F))
    return out.reshape(B, G, F)


# speedup vs baseline: 1.0037x; 1.0037x over previous
import functools

import numpy as np
import jax
import jax.numpy as jnp
from jax import lax
from jax.experimental import pallas as pl
from jax.experimental.pallas import tpu as pltpu
from jax.experimental.pallas import tpu_sc as plsc

B, H, N, F = 8, 12, 576, 768
G = 16
L = 16
NC, NS = 2, 16
NW = NC * NS
JCH = N // L
NEG = np.float32(-3.0e38)

HSC = 5
HTC = H - HSC
RCH = 72
RPS = N // RCH
NSLAB = B * HSC
NIT = NSLAB * RPS
IPT = NIT // NW
JB = 18

CH = 1
NBUF = 8


def _sc_reduce(attn):
    mesh = plsc.VectorSubcoreMesh(core_axis_name="c", subcore_axis_name="s")

    @functools.partial(
        pl.kernel,
        out_type=jax.ShapeDtypeStruct((B, NW, N), jnp.float32),
        mesh=mesh,
        scratch_types=[
            pltpu.VMEM((2, RCH, N), jnp.float32),
            pltpu.VMEM((B, N), jnp.float32),
            pltpu.SemaphoreType.DMA((2,)),
        ],
    )
    def k(a_hbm, parts_hbm, buf, acc, sems):
        wid = lax.axis_index("s") * NC + lax.axis_index("c")
        zero = jnp.zeros((L,), jnp.float32)
        for bb in range(B):
            for j in range(JCH):
                acc[bb, pl.ds(j * L, L)] = zero

        def copy_item(q, parity):
            it = wid + q * NW
            s = it // RPS
            c = it - s * RPS
            bs = s // HSC
            hs = HTC + (s - bs * HSC)
            return pltpu.make_async_copy(
                a_hbm.at[bs, hs, pl.ds(c * RCH, RCH)],
                buf.at[parity], sems.at[parity])

        copy_item(0, 0).start()
        for q in range(IPT):
            parity = q % 2
            if q + 1 < IPT:
                copy_item(q + 1, 1 - parity).start()
            copy_item(q, parity).wait()
            it = wid + q * NW
            b = it // (HSC * RPS)
            for j0 in range(0, JCH, JB):
                def body(r, accs):
                    return tuple(
                        a + buf[parity, r, pl.ds((j0 + j) * L, L)]
                        for j, a in enumerate(accs))

                accs = lax.fori_loop(0, RCH, body,
                                     tuple(zero for _ in range(JB)))
                for j, a in enumerate(accs):
                    off = (j0 + j) * L
                    acc[b, pl.ds(off, L)] = acc[b, pl.ds(off, L)] + a
        pltpu.sync_copy(acc, parts_hbm.at[:, wid])

    return k(attn)


def _tc_scores(attn):
    hpc = HTC // CH

    def body(a_hbm, o_ref, bufs, sems):
        def start(b, hh):
            slot = (b * hpc + hh) % NBUF
            pltpu.make_async_copy(a_hbm.at[b, pl.ds(hh * CH, CH)],
                                  bufs.at[slot], sems.at[slot]).start()

        for i in range(NBUF):
            start(i // hpc, i % hpc)
        for b in range(B):
            acc = jnp.zeros((N,), jnp.float32)
            for hh in range(hpc):
                i = b * hpc + hh
                slot = i % NBUF
                pltpu.make_async_copy(a_hbm.at[b, pl.ds(hh * CH, CH)],
                                      bufs.at[slot], sems.at[slot]).wait()
                acc = acc + jnp.sum(bufs[slot], axis=(0, 1))
                if i + NBUF < B * hpc:
                    j = i + NBUF
                    start(j // hpc, j % hpc)
            o_ref[b] = acc

    return pl.pallas_call(
        body,
        in_specs=[pl.BlockSpec(memory_space=pltpu.HBM)],
        out_specs=pl.BlockSpec(memory_space=pltpu.VMEM),
        out_shape=jax.ShapeDtypeStruct((B, N), jnp.float32),
        scratch_shapes=[pltpu.VMEM((NBUF, CH, N, N), jnp.float32),
                        pltpu.SemaphoreType.DMA((NBUF,))],
    )(attn)


def _topk_gather(tc_scores, parts, x2):
    mesh = plsc.VectorSubcoreMesh(core_axis_name="c", subcore_axis_name="s")

    @functools.partial(
        pl.kernel,
        out_type=jax.ShapeDtypeStruct((B * G, F), jnp.float32),
        mesh=mesh,
        scratch_types=[
            pltpu.VMEM((N,), jnp.float32),
            pltpu.VMEM((NW, N), jnp.float32),
            pltpu.VMEM((G,), jnp.int32),
            pltpu.VMEM((G, F), jnp.float32),
            pltpu.SemaphoreType.DMA,
        ],
    )
    def k(tc_hbm, parts_hbm, x_hbm, out_hbm, s_v, p_v, idx_v, rows_v, sem):
        wid = lax.axis_index("s") * NC + lax.axis_index("c")

        @pl.when(wid < B)
        def _():
            b = wid
            pltpu.sync_copy(tc_hbm.at[b], s_v)
            pltpu.sync_copy(parts_hbm.at[b], p_v)
            lanes = lax.iota(jnp.int32, L)

            for j0 in range(0, JCH, JB):
                def body(t, accs):
                    return tuple(a + p_v[t, pl.ds((j0 + j) * L, L)]
                                 for j, a in enumerate(accs))

                accs = lax.fori_loop(
                    0, NW, body,
                    tuple(s_v[pl.ds((j0 + j) * L, L)] for j in range(JB)))
                for j, a in enumerate(accs):
                    s_v[pl.ds((j0 + j) * L, L)] = a

            def outer(k_i, topk):
                def scan(j, c):
                    bv, bi = c
                    v = s_v[pl.ds(j * L, L)]
                    take = v > bv
                    return (jnp.where(take, v, bv),
                            jnp.where(take, j * L + lanes, bi))

                bv, bi = lax.fori_loop(
                    0, JCH, scan,
                    (jnp.full((L,), NEG, jnp.float32),
                     jnp.zeros((L,), jnp.int32)))
                best, besti = bv[0], bi[0]
                for i in range(1, L):
                    vi, ni = bv[i], bi[i]
                    upd = (vi > best) | ((vi == best) & (ni < besti))
                    best = jnp.where(upd, vi, best)
                    besti = jnp.where(upd, ni, besti)
                cb = (besti // L) * L
                cur = s_v[pl.ds(cb, L)]
                s_v[pl.ds(cb, L)] = jnp.where(lanes == besti - cb, NEG, cur)
                return jnp.where(lanes == k_i, besti, topk)

            topk = lax.fori_loop(0, G, outer, jnp.zeros((L,), jnp.int32))
            idx_v[...] = topk + b * N
            pltpu.async_copy(x_hbm.at[idx_v], rows_v, sem).wait()
            pltpu.sync_copy(rows_v, out_hbm.at[pl.ds(b * G, G)])

    return k(tc_scores, parts, x2)


def kernel(x, attn_weights):
    parts = _sc_reduce(attn_weights)
    tc = _tc_scores(attn_weights)
    out = _topk_gather(tc, parts, x.reshape(B * N, F))
    return out.reshape(B, G, F)


# speedup vs baseline: 1.1767x; 1.1724x over previous
import functools

import numpy as np
import jax
import jax.numpy as jnp
from jax import lax
from jax.experimental import pallas as pl
from jax.experimental.pallas import tpu as pltpu
from jax.experimental.pallas import tpu_sc as plsc

B, H, N, F = 8, 12, 576, 768
G = 16
HB = 6
L = 16
NC, NS = 2, 16
NEG = np.float32(-3.0e38)


CH = 2
NBUF = 8


def _scores(attn):
    nch = B * H // CH
    hpc = H // CH
    attn4 = attn.reshape(nch, CH, N, N)

    def body(a_hbm, o_ref, bufs, sems):
        def start(i):
            slot = i % NBUF
            pltpu.make_async_copy(a_hbm.at[i], bufs.at[slot],
                                  sems.at[slot]).start()

        for i in range(NBUF):
            start(i)
        for b in range(B):
            acc = jnp.zeros((N,), jnp.float32)
            for hh in range(hpc):
                i = b * hpc + hh
                slot = i % NBUF
                pltpu.make_async_copy(a_hbm.at[i], bufs.at[slot],
                                      sems.at[slot]).wait()
                acc = acc + jnp.sum(bufs[slot], axis=(0, 1))
                if i + NBUF < nch:
                    start(i + NBUF)
            o_ref[b] = acc

    return pl.pallas_call(
        body,
        in_specs=[pl.BlockSpec(memory_space=pltpu.HBM)],
        out_specs=pl.BlockSpec(memory_space=pltpu.VMEM),
        out_shape=jax.ShapeDtypeStruct((B, N), jnp.float32),
        scratch_shapes=[pltpu.VMEM((NBUF, CH, N, N), jnp.float32),
                        pltpu.SemaphoreType.DMA((NBUF,))],
    )(attn4)


def _topk_gather(scores, x2):
    mesh = plsc.VectorSubcoreMesh(core_axis_name="c", subcore_axis_name="s",
                                  num_cores=1)

    @functools.partial(
        pl.kernel,
        out_type=jax.ShapeDtypeStruct((B * G, F), jnp.float32),
        mesh=mesh,
        scratch_types=[
            pltpu.VMEM((N,), jnp.float32),
            pltpu.VMEM((G,), jnp.int32),
            pltpu.VMEM((G, F), jnp.float32),
            pltpu.SemaphoreType.DMA,
        ],
    )
    def k(scores_hbm, x_hbm, out_hbm, s_v, idx_v, rows_v, sem):
        wid = lax.axis_index("s")

        @pl.when(wid < B)
        def _():
            b = wid
            pltpu.sync_copy(scores_hbm.at[b], s_v)
            lanes = lax.iota(jnp.int32, L)

            def outer(k_i, topk):
                def scan(j, c):
                    bv, bi = c
                    v = s_v[pl.ds(j * L, L)]
                    take = v > bv
                    return (jnp.where(take, v, bv),
                            jnp.where(take, j * L + lanes, bi))

                bv, bi = lax.fori_loop(
                    0, N // L, scan,
                    (jnp.full((L,), NEG, jnp.float32),
                     jnp.zeros((L,), jnp.int32)))
                best, besti = bv[0], bi[0]
                for i in range(1, L):
                    vi, ni = bv[i], bi[i]
                    upd = (vi > best) | ((vi == best) & (ni < besti))
                    best = jnp.where(upd, vi, best)
                    besti = jnp.where(upd, ni, besti)
                cb = (besti // L) * L
                cur = s_v[pl.ds(cb, L)]
                s_v[pl.ds(cb, L)] = jnp.where(lanes == besti - cb, NEG, cur)
                return jnp.where(lanes == k_i, besti, topk)

            topk = lax.fori_loop(0, G, outer, jnp.zeros((L,), jnp.int32))
            idx_v[...] = topk + b * N
            pltpu.async_copy(x_hbm.at[idx_v], rows_v, sem).wait()
            pltpu.sync_copy(rows_v, out_hbm.at[pl.ds(b * G, G)])

    return k(scores, x2)


def kernel(x, attn_weights):
    scores = _scores(attn_weights)
    out = _topk_gather(scores, x.reshape(B * N, F))
    return out.reshape(B, G, F)


# speedup vs baseline: 1.2199x; 1.0368x over previous
import functools

import numpy as np
import jax
import jax.numpy as jnp
from jax import lax
from jax.experimental import pallas as pl
from jax.experimental.pallas import tpu as pltpu
from jax.experimental.pallas import tpu_sc as plsc

B, H, N, F = 8, 12, 576, 768
G = 16
HB = 6
L = 16
NC, NS = 2, 16
NEG = np.float32(-3.0e38)


CH = 2
NBUF = 8


def _scores(attn):
    nch = B * H // CH
    hpc = H // CH
    attn4 = attn.reshape(nch, CH, N, N)

    def body(a_hbm, o_ref, bufs, sems):
        def start(i):
            slot = i % NBUF
            pltpu.make_async_copy(a_hbm.at[i], bufs.at[slot],
                                  sems.at[slot]).start()

        for i in range(NBUF):
            start(i)
        for b in range(B):
            acc = jnp.zeros((N,), jnp.float32)
            for hh in range(hpc):
                i = b * hpc + hh
                slot = i % NBUF
                pltpu.make_async_copy(a_hbm.at[i], bufs.at[slot],
                                      sems.at[slot]).wait()
                acc = acc + jnp.sum(bufs[slot], axis=(0, 1))
                if i + NBUF < nch:
                    start(i + NBUF)
            o_ref[b] = acc

    return pl.pallas_call(
        body,
        in_specs=[pl.BlockSpec(memory_space=pltpu.HBM)],
        out_specs=pl.BlockSpec(memory_space=pltpu.VMEM),
        out_shape=jax.ShapeDtypeStruct((B, N), jnp.float32),
        scratch_shapes=[pltpu.VMEM((NBUF, CH, N, N), jnp.float32),
                        pltpu.SemaphoreType.DMA((NBUF,))],
    )(attn4)


def _topk_gather(scores, x2):
    mesh = plsc.VectorSubcoreMesh(core_axis_name="c", subcore_axis_name="s",
                                  num_cores=1)

    @functools.partial(
        pl.kernel,
        out_type=jax.ShapeDtypeStruct((B * G, F), jnp.float32),
        mesh=mesh,
        scratch_types=[
            pltpu.VMEM((N,), jnp.float32),
            pltpu.VMEM((G,), jnp.int32),
            pltpu.VMEM((G, F), jnp.float32),
            pltpu.SemaphoreType.DMA,
        ],
    )
    def k(scores_hbm, x_hbm, out_hbm, s_v, idx_v, rows_v, sem):
        wid = lax.axis_index("s")

        @pl.when(wid < B)
        def _():
            b = wid
            pltpu.sync_copy(scores_hbm.at[b], s_v)
            lanes = lax.iota(jnp.int32, L)

            def outer(k_i, topk):
                bv = s_v[pl.ds(0, L)]
                bi = lanes
                for j in range(1, N // L):
                    v = s_v[pl.ds(j * L, L)]
                    take = v > bv
                    bv = jnp.where(take, v, bv)
                    bi = jnp.where(take, j * L + lanes, bi)

                def comb(a, c):
                    va, na = a
                    vc, nc2 = c
                    t = (vc > va) | ((vc == va) & (nc2 < na))
                    return (jnp.where(t, vc, va), jnp.where(t, nc2, na))

                cur = [(bv[i], bi[i]) for i in range(L)]
                while len(cur) > 1:
                    cur = [comb(cur[i], cur[i + 1])
                           for i in range(0, len(cur), 2)]
                best, besti = cur[0]
                cb = (besti // L) * L
                cur = s_v[pl.ds(cb, L)]
                s_v[pl.ds(cb, L)] = jnp.where(lanes == besti - cb, NEG, cur)
                return jnp.where(lanes == k_i, besti, topk)

            topk = lax.fori_loop(0, G, outer, jnp.zeros((L,), jnp.int32))
            idx_v[...] = topk + b * N
            pltpu.async_copy(x_hbm.at[idx_v], rows_v, sem).wait()
            pltpu.sync_copy(rows_v, out_hbm.at[pl.ds(b * G, G)])

    return k(scores, x2)


def kernel(x, attn_weights):
    scores = _scores(attn_weights)
    out = _topk_gather(scores, x.reshape(B * N, F))
    return out.reshape(B, G, F)
